# Initial kernel scaffold; baseline (speedup 1.0000x reference)
import jax
import jax.numpy as jnp
from jax.experimental import pallas as pl


def kernel(x, edge_index, edge_label_index, Wa, ba, W1l, b1, W1r, W2l, b2, W2r):
    raise NotImplementedError("write your pallas kernel here")


# speedup vs baseline: 1.3924x; 1.3924x over previous
import functools

import jax
import jax.numpy as jnp
from jax import lax
from jax.experimental import pallas as pl
from jax.experimental.pallas import tpu as pltpu
from jax.experimental.pallas import tpu_sc as plsc

N_NODES = 10000
HID = 128
NC, NS, LANES = 2, 16, 16
NW = NC * NS

IW = 128
N_PAD = 10112
RPT = N_PAD // NS
DST_PAD = N_NODES + 8

E = 320000
E_PAD = 327680
EROWS_T = E_PAD // (NW * IW)
ER_TOT = E_PAD // IW

NLBL = 200000
L_PAD = 229376
LROWS_T = L_PAD // (NW * IW)
LR_TOT = L_PAD // IW
LPT = L_PAD // NW
CW = 16

_MESH = plsc.VectorSubcoreMesh(
    core_axis_name="c", subcore_axis_name="s", num_cores=NC, num_subcores=NS)


def _tc_linear(x, w, b):
    m, k = x.shape
    n = w.shape[1]
    br = 1000

    def body(x_ref, w_ref, b_ref, o_ref):
        o_ref[...] = (
            jnp.dot(x_ref[...], w_ref[...], preferred_element_type=jnp.float32)
            + b_ref[...])

    return pl.pallas_call(
        body,
        grid=(m // br,),
        in_specs=[
            pl.BlockSpec((br, k), lambda i: (i, 0)),
            pl.BlockSpec((k, n), lambda i: (0, 0)),
            pl.BlockSpec((1, n), lambda i: (0, 0)),
        ],
        out_specs=pl.BlockSpec((br, n), lambda i: (i, 0)),
        out_shape=jax.ShapeDtypeStruct((m, n), jnp.float32),
    )(x, w, b.reshape(1, n))


def _tc_combine(parts, cnt1, h, wl, b, wr, relu):
    br = 1000

    def body(p_ref, c_ref, h_ref, wl_ref, wr_ref, b_ref, o_ref):
        agg = p_ref[0] + p_ref[1]
        mean = agg / jnp.maximum(c_ref[...], 1.0)
        out = (
            jnp.dot(mean, wl_ref[...], preferred_element_type=jnp.float32)
            + jnp.dot(h_ref[...], wr_ref[...], preferred_element_type=jnp.float32)
            + b_ref[...])
        if relu:
            out = jnp.maximum(out, 0.0)
        o_ref[...] = out

    return pl.pallas_call(
        body,
        grid=(N_NODES // br,),
        in_specs=[
            pl.BlockSpec((NC, br, HID), lambda i: (0, i, 0)),
            pl.BlockSpec((br, 1), lambda i: (i, 0)),
            pl.BlockSpec((br, HID), lambda i: (i, 0)),
            pl.BlockSpec((HID, HID), lambda i: (0, 0)),
            pl.BlockSpec((HID, HID), lambda i: (0, 0)),
            pl.BlockSpec((1, HID), lambda i: (0, 0)),
        ],
        out_specs=pl.BlockSpec((br, HID), lambda i: (i, 0)),
        out_shape=jax.ShapeDtypeStruct((N_NODES, HID), jnp.float32),
    )(parts, cnt1, h, wl, wr, b.reshape(1, HID))


def _make_sc_agg(with_cnt):
    out_type = jax.ShapeDtypeStruct((NC, N_PAD, HID), jnp.float32)
    scratch = [
        pltpu.VMEM((EROWS_T, IW), jnp.int32),
        pltpu.VMEM((EROWS_T, IW), jnp.int32),
        pltpu.VMEM((IW, HID), jnp.float32),
        pltpu.VMEM_SHARED((N_PAD, HID), jnp.float32),
    ]
    if with_cnt:
        out_type = (out_type, jax.ShapeDtypeStruct((NC * N_PAD,), jnp.float32))
        scratch += [
            pltpu.VMEM((IW,), jnp.float32),
            pltpu.VMEM((RPT,), jnp.float32),
            pltpu.VMEM_SHARED((N_PAD,), jnp.float32),
        ]

    def body(*refs):
        if with_cnt:
            (h_hbm, src_hbm, dst_hbm, zacc_hbm,
             part_hbm, cntp_hbm, src_v, dst_v, rows_v, acc_sh,
             ones_v, cbuf_v, cnt_sh) = refs
        else:
            (h_hbm, src_hbm, dst_hbm, zacc_hbm,
             part_hbm, src_v, dst_v, rows_v, acc_sh) = refs
        cid = lax.axis_index("c")
        sid = lax.axis_index("s")
        wid = cid * NS + sid
        r0 = sid * RPT
        pltpu.sync_copy(zacc_hbm.at[pl.ds(r0, RPT)], acc_sh.at[pl.ds(r0, RPT)])
        if with_cnt:
            z16 = jnp.zeros((LANES,), jnp.float32)
            o16 = jnp.ones((LANES,), jnp.float32)
            for k in range(IW // LANES):
                ones_v[pl.ds(k * LANES, LANES)] = o16
            for k in range(RPT // LANES):
                cbuf_v[pl.ds(k * LANES, LANES)] = z16
            cbuf_v[pl.ds(RPT - LANES, LANES)] = z16
            pltpu.sync_copy(cbuf_v, cnt_sh.at[pl.ds(r0, RPT)])
        plsc.subcore_barrier()
        pltpu.sync_copy(src_hbm.at[pl.ds(wid * EROWS_T, EROWS_T)], src_v)
        pltpu.sync_copy(dst_hbm.at[pl.ds(wid * EROWS_T, EROWS_T)], dst_v)

        def step(j, carry):
            pltpu.sync_copy(h_hbm.at[src_v.at[j]], rows_v)
            pltpu.sync_copy(rows_v, acc_sh.at[dst_v.at[j]], add=True)
            if with_cnt:
                pltpu.sync_copy(ones_v, cnt_sh.at[dst_v.at[j]], add=True)
            return carry

        lax.fori_loop(0, EROWS_T, step, 0)
        plsc.subcore_barrier()
        pltpu.sync_copy(acc_sh.at[pl.ds(r0, RPT)],
                        part_hbm.at[cid, pl.ds(r0, RPT)])
        if with_cnt:
            pltpu.sync_copy(cnt_sh.at[pl.ds(r0, RPT)], cbuf_v)
            pltpu.sync_copy(cbuf_v, cntp_hbm.at[pl.ds(cid * N_PAD + r0, RPT)])

    return pl.kernel(body, out_type=out_type, mesh=_MESH,
                     scratch_types=scratch)


_sc_agg_cnt = _make_sc_agg(True)
_sc_agg = _make_sc_agg(False)


@functools.partial(
    pl.kernel,
    out_type=jax.ShapeDtypeStruct((L_PAD * LANES,), jnp.float32),
    mesh=_MESH,
    scratch_types=[
        pltpu.VMEM((LROWS_T, IW), jnp.int32),
        pltpu.VMEM((LROWS_T, IW), jnp.int32),
        pltpu.VMEM((IW, HID), jnp.float32),
        pltpu.VMEM((IW, HID), jnp.float32),
        pltpu.VMEM((IW * LANES,), jnp.float32),
    ],
)
def _sc_dots(h2_hbm, e1_hbm, e2_hbm, out_hbm, i1_v, i2_v, a_v, b_v, o_v):
    cid = lax.axis_index("c")
    sid = lax.axis_index("s")
    wid = cid * NS + sid
    pltpu.sync_copy(e1_hbm.at[pl.ds(wid * LROWS_T, LROWS_T)], i1_v)
    pltpu.sync_copy(e2_hbm.at[pl.ds(wid * LROWS_T, LROWS_T)], i2_v)

    def step(j, carry):
        pltpu.sync_copy(h2_hbm.at[i1_v.at[j]], a_v)
        pltpu.sync_copy(h2_hbm.at[i2_v.at[j]], b_v)
        for row in range(IW):
            v = a_v[row, pl.ds(0, LANES)] * b_v[row, pl.ds(0, LANES)]
            for k in range(1, HID // LANES):
                v = v + (a_v[row, pl.ds(k * LANES, LANES)]
                         * b_v[row, pl.ds(k * LANES, LANES)])
            o_v[pl.ds(row * LANES, LANES)] = v
        pltpu.sync_copy(
            o_v,
            out_hbm.at[pl.ds((wid * LROWS_T + j) * IW * LANES, IW * LANES)])
        return carry

    lax.fori_loop(0, LROWS_T, step, 0)


def _tc_lanesum(partials):
    p = partials.reshape(L_PAD * LANES // IW, IW)
    g = jnp.equal(
        lax.broadcasted_iota(jnp.int32, (IW, IW // LANES), 0) // LANES,
        lax.broadcasted_iota(jnp.int32, (IW, IW // LANES), 1),
    ).astype(jnp.float32)
    br = 4096

    def body(p_ref, g_ref, o_ref):
        o_ref[...] = jnp.dot(p_ref[...], g_ref[...],
                             preferred_element_type=jnp.float32)

    out = pl.pallas_call(
        body,
        grid=(p.shape[0] // br,),
        in_specs=[
            pl.BlockSpec((br, IW), lambda i: (i, 0)),
            pl.BlockSpec((IW, IW // LANES), lambda i: (0, 0)),
        ],
        out_specs=pl.BlockSpec((br, IW // LANES), lambda i: (i, 0)),
        out_shape=jax.ShapeDtypeStruct((p.shape[0], IW // LANES), jnp.float32),
    )(p, g)
    return out.reshape(L_PAD)


def _pad_reshape(idx, total, pad_val, rows):
    pad = jnp.full((total - idx.shape[0],), pad_val, jnp.int32)
    return jnp.concatenate([idx.astype(jnp.int32), pad]).reshape(rows, IW)


def kernel(x, edge_index, edge_label_index, Wa, ba, W1l, b1, W1r, W2l, b2, W2r):
    src = _pad_reshape(edge_index[0], E_PAD, 0, ER_TOT)
    dst = _pad_reshape(edge_index[1], E_PAD, DST_PAD, ER_TOT)
    e1 = _pad_reshape(edge_label_index[0], L_PAD, 0, LR_TOT)
    e2 = _pad_reshape(edge_label_index[1], L_PAD, 0, LR_TOT)
    zacc = jnp.zeros((N_PAD, HID), jnp.float32)

    h0 = _tc_linear(x, Wa, ba)
    part1, cntp = _sc_agg_cnt(h0, src, dst, zacc)
    cnt1 = (cntp.reshape(NC, N_PAD).sum(axis=0))[:, None]
    h1 = _tc_combine(part1, cnt1, h0, W1l, b1, W1r, True)
    part2 = _sc_agg(h1, src, dst, zacc)
    h2 = _tc_combine(part2, cnt1, h1, W2l, b2, W2r, False)
    partials = _sc_dots(h2, e1, e2)
    dots = _tc_lanesum(partials)
    return dots[:NLBL]


# speedup vs baseline: 3.5206x; 2.5284x over previous
import functools

import jax
import jax.numpy as jnp
from jax import lax
from jax.experimental import pallas as pl
from jax.experimental.pallas import tpu as pltpu
from jax.experimental.pallas import tpu_sc as plsc

N_NODES = 10000
HID = 128
NC, NS, LANES = 2, 16, 16
NW = NC * NS

IW = 128
N_PAD = 10112
RPT = N_PAD // NS
DST_PAD = N_NODES + 8

E = 320000
E_PAD = 327680
EROWS_T = E_PAD // (NW * IW)
ER_TOT = E_PAD // IW

NLBL = 200000
L_PAD = 200704
LROWS_T = L_PAD // (NW * IW)
LPT = L_PAD // NW

_MESH = plsc.VectorSubcoreMesh(
    core_axis_name="c", subcore_axis_name="s", num_cores=NC, num_subcores=NS)


def _tc_linear(x, w, b):
    m, k = x.shape
    n = w.shape[1]
    br = 1000

    def body(x_ref, w_ref, b_ref, o_ref):
        o_ref[...] = (
            jnp.dot(x_ref[...], w_ref[...], preferred_element_type=jnp.float32)
            + b_ref[...])

    return pl.pallas_call(
        body,
        grid=(m // br,),
        in_specs=[
            pl.BlockSpec((br, k), lambda i: (i, 0)),
            pl.BlockSpec((k, n), lambda i: (0, 0)),
            pl.BlockSpec((1, n), lambda i: (0, 0)),
        ],
        out_specs=pl.BlockSpec((br, n), lambda i: (i, 0)),
        out_shape=jax.ShapeDtypeStruct((m, n), jnp.float32),
    )(x, w, b.reshape(1, n))


def _tc_combine(parts, cnt1, h, wl, b, wr, relu):
    br = 1000

    def body(p_ref, c_ref, h_ref, wl_ref, wr_ref, b_ref, o_ref):
        agg = p_ref[0] + p_ref[1]
        mean = agg / jnp.maximum(c_ref[...], 1.0)
        out = (
            jnp.dot(mean, wl_ref[...], preferred_element_type=jnp.float32)
            + jnp.dot(h_ref[...], wr_ref[...], preferred_element_type=jnp.float32)
            + b_ref[...])
        if relu:
            out = jnp.maximum(out, 0.0)
        o_ref[...] = out

    return pl.pallas_call(
        body,
        grid=(N_NODES // br,),
        in_specs=[
            pl.BlockSpec((NC, br, HID), lambda i: (0, i, 0)),
            pl.BlockSpec((br, 1), lambda i: (i, 0)),
            pl.BlockSpec((br, HID), lambda i: (i, 0)),
            pl.BlockSpec((HID, HID), lambda i: (0, 0)),
            pl.BlockSpec((HID, HID), lambda i: (0, 0)),
            pl.BlockSpec((1, HID), lambda i: (0, 0)),
        ],
        out_specs=pl.BlockSpec((br, HID), lambda i: (i, 0)),
        out_shape=jax.ShapeDtypeStruct((N_NODES, HID), jnp.float32),
    )(parts, cnt1, h, wl, wr, b.reshape(1, HID))


def _make_sc_agg(with_cnt):
    out_type = jax.ShapeDtypeStruct((NC, N_PAD, HID), jnp.float32)
    scratch = [
        pltpu.VMEM((EROWS_T, IW), jnp.int32),
        pltpu.VMEM((2, IW), jnp.int32),
        pltpu.VMEM((2, IW), jnp.int32),
        pltpu.VMEM((2 * IW, HID), jnp.float32),
        pltpu.SemaphoreType.DMA,
        pltpu.VMEM_SHARED((N_PAD, HID), jnp.float32),
    ]
    if with_cnt:
        out_type = (out_type, jax.ShapeDtypeStruct((NC * N_PAD,), jnp.float32))
        scratch += [
            pltpu.VMEM((IW,), jnp.float32),
            pltpu.VMEM((RPT,), jnp.float32),
            pltpu.VMEM_SHARED((N_PAD,), jnp.float32),
        ]

    def body(*refs):
        if with_cnt:
            (h_hbm, epk_hbm, zacc_hbm,
             part_hbm, cntp_hbm, pk_v, src2_v, dst2_v, rows_v,
             gs, acc_sh,
             ones_v, cbuf_v, cnt_sh) = refs
        else:
            (h_hbm, epk_hbm, zacc_hbm,
             part_hbm, pk_v, src2_v, dst2_v, rows_v,
             gs, acc_sh) = refs
        cid = lax.axis_index("c")
        sid = lax.axis_index("s")
        wid = cid * NS + sid
        r0 = sid * RPT
        pltpu.sync_copy(zacc_hbm.at[pl.ds(r0, RPT)], acc_sh.at[pl.ds(r0, RPT)])
        if with_cnt:
            z16 = jnp.zeros((LANES,), jnp.float32)
            o16 = jnp.ones((LANES,), jnp.float32)
            for k in range(IW // LANES):
                ones_v[pl.ds(k * LANES, LANES)] = o16
            for k in range(RPT // LANES):
                cbuf_v[pl.ds(k * LANES, LANES)] = z16
            cbuf_v[pl.ds(RPT - LANES, LANES)] = z16
            pltpu.sync_copy(cbuf_v, cnt_sh.at[pl.ds(r0, RPT)])
        plsc.subcore_barrier()
        pltpu.sync_copy(epk_hbm.at[wid], pk_v)

        def unpack(j, slot):
            for k in range(IW // LANES):
                p = pk_v[j, pl.ds(k * LANES, LANES)]
                src2_v[slot, pl.ds(k * LANES, LANES)] = (
                    jnp.bitwise_and(p, 0xFFFF))
                dst2_v[slot, pl.ds(k * LANES, LANES)] = (
                    lax.shift_right_logical(p, 16))

        unpack(0, 0)
        pltpu.async_copy(h_hbm.at[src2_v.at[0]],
                         rows_v.at[pl.ds(0, IW)], gs)

        @pl.loop(0, EROWS_T)
        def _pipeline(j):
            p = j & 1
            po = p * IW
            qo = IW - po
            pltpu.make_async_copy(h_hbm.at[src2_v.at[p]],
                                  rows_v.at[pl.ds(po, IW)], gs).wait()

            @pl.when(j + 1 < EROWS_T)
            def _prefetch():
                unpack(j + 1, 1 - p)
                pltpu.async_copy(h_hbm.at[src2_v.at[1 - p]],
                                 rows_v.at[pl.ds(qo, IW)], gs)

            pltpu.sync_copy(rows_v.at[pl.ds(po, IW)],
                            acc_sh.at[dst2_v.at[p]], add=True)
            if with_cnt:
                pltpu.sync_copy(ones_v, cnt_sh.at[dst2_v.at[p]], add=True)
        plsc.subcore_barrier()
        pltpu.sync_copy(acc_sh.at[pl.ds(r0, RPT)],
                        part_hbm.at[cid, pl.ds(r0, RPT)])
        if with_cnt:
            pltpu.sync_copy(cnt_sh.at[pl.ds(r0, RPT)], cbuf_v)
            pltpu.sync_copy(cbuf_v, cntp_hbm.at[pl.ds(cid * N_PAD + r0, RPT)])

    return pl.kernel(body, out_type=out_type, mesh=_MESH,
                     scratch_types=scratch)


_sc_agg_cnt = _make_sc_agg(True)
_sc_agg = _make_sc_agg(False)


@functools.partial(
    pl.kernel,
    out_type=(jax.ShapeDtypeStruct((L_PAD, HID), jnp.float32),
              jax.ShapeDtypeStruct((L_PAD, HID), jnp.float32)),
    mesh=_MESH,
    scratch_types=[
        pltpu.VMEM((LROWS_T, IW), jnp.int32),
        pltpu.VMEM((LROWS_T, IW), jnp.int32),
        pltpu.VMEM((IW, HID), jnp.float32),
        pltpu.VMEM((IW, HID), jnp.float32),
        pltpu.VMEM((IW, HID), jnp.float32),
        pltpu.VMEM((IW, HID), jnp.float32),
        pltpu.SemaphoreType.DMA,
        pltpu.SemaphoreType.DMA,
        pltpu.SemaphoreType.DMA,
        pltpu.SemaphoreType.DMA,
    ],
)
def _sc_gather_pairs(h2_hbm, e1_hbm, e2_hbm, a_hbm, b_hbm,
                     i1_v, i2_v, a0_v, a1_v, b0_v, b1_v, g0, g1, w0, w1):
    cid = lax.axis_index("c")
    sid = lax.axis_index("s")
    wid = cid * NS + sid
    pltpu.sync_copy(e1_hbm.at[wid], i1_v)
    pltpu.sync_copy(e2_hbm.at[wid], i2_v)
    bufs = ((a0_v, b0_v, g0, w0), (a1_v, b1_v, g1, w1))

    @pl.loop(0, LROWS_T - 1, step=2)
    def _pipeline(t):
        gd = []
        for b in range(2):
            a_v, b_v, gs, _ = bufs[b]
            gd.append(pltpu.async_copy(h2_hbm.at[i1_v.at[t + b]], a_v, gs))
            gd.append(pltpu.async_copy(h2_hbm.at[i2_v.at[t + b]], b_v, gs))
        wd = []
        for b in range(2):
            a_v, b_v, _, ws = bufs[b]
            base = (wid * LROWS_T + t + b) * IW
            gd[2 * b].wait()
            gd[2 * b + 1].wait()
            wd.append(pltpu.async_copy(a_v, a_hbm.at[pl.ds(base, IW)], ws))
            wd.append(pltpu.async_copy(b_v, b_hbm.at[pl.ds(base, IW)], ws))
        for d in wd:
            d.wait()

    j = LROWS_T - 1
    a_v, b_v, gs, ws = bufs[0]
    pltpu.sync_copy(h2_hbm.at[i1_v.at[j]], a_v)
    pltpu.sync_copy(h2_hbm.at[i2_v.at[j]], b_v)
    base = (wid * LROWS_T + j) * IW
    pltpu.sync_copy(a_v, a_hbm.at[pl.ds(base, IW)])
    pltpu.sync_copy(b_v, b_hbm.at[pl.ds(base, IW)])


def _tc_pair_dot(arows, brows):
    br = 4096

    def body(a_ref, b_ref, o_ref):
        o_ref[...] = jnp.sum(a_ref[...] * b_ref[...], axis=1, keepdims=True)

    return pl.pallas_call(
        body,
        grid=(L_PAD // br,),
        in_specs=[
            pl.BlockSpec((br, HID), lambda i: (i, 0)),
            pl.BlockSpec((br, HID), lambda i: (i, 0)),
        ],
        out_specs=pl.BlockSpec((br, 1), lambda i: (i, 0)),
        out_shape=jax.ShapeDtypeStruct((L_PAD, 1), jnp.float32),
    )(arows, brows)


def _pad_reshape(idx, total, pad_val, rows):
    pad = jnp.full((total - idx.shape[0],), pad_val, jnp.int32)
    return jnp.concatenate([idx.astype(jnp.int32), pad]).reshape(rows, IW)


def kernel(x, edge_index, edge_label_index, Wa, ba, W1l, b1, W1r, W2l, b2, W2r):
    src = _pad_reshape(edge_index[0], E_PAD, 0, NW * EROWS_T)
    dst = _pad_reshape(edge_index[1], E_PAD, DST_PAD, NW * EROWS_T)
    epk = (src | (dst << 16)).reshape(NW, EROWS_T, IW)
    e1 = _pad_reshape(edge_label_index[0], L_PAD, 0, NW * LROWS_T
                      ).reshape(NW, LROWS_T, IW)
    e2 = _pad_reshape(edge_label_index[1], L_PAD, 0, NW * LROWS_T
                      ).reshape(NW, LROWS_T, IW)
    zacc = jnp.zeros((N_PAD, HID), jnp.float32)

    h0 = _tc_linear(x, Wa, ba)
    part1, cntp = _sc_agg_cnt(h0, epk, zacc)
    cnt1 = (cntp.reshape(NC, N_PAD).sum(axis=0))[:, None]
    h1 = _tc_combine(part1, cnt1, h0, W1l, b1, W1r, True)
    part2 = _sc_agg(h1, epk, zacc)
    h2 = _tc_combine(part2, cnt1, h1, W2l, b2, W2r, False)
    arows, brows = _sc_gather_pairs(h2, e1, e2)
    dots = _tc_pair_dot(arows, brows)
    return dots[:NLBL, 0]
